# Initial kernel scaffold; baseline (speedup 1.0000x reference)
import jax
import jax.numpy as jnp
from jax.experimental import pallas as pl


def kernel(x, table):
    raise NotImplementedError("write your pallas kernel here")


# speedup vs baseline: 134.5295x; 134.5295x over previous
import functools

import jax
import jax.numpy as jnp
from jax import lax
from jax.experimental import pallas as pl
from jax.experimental.pallas import tpu as pltpu
from jax.experimental.pallas import tpu_sc as plsc

ROWS, COLS = 16384, 200
N = ROWS * COLS
NC, NS, L = 2, 16, 16
NW = NC * NS
PER_W = N // NW
CHUNK = 25_600
NCHUNK = PER_W // CHUNK
WPC = CHUNK // 4


def _sc_lut_call(xw, tbl32):
    mesh = plsc.VectorSubcoreMesh(core_axis_name="c", subcore_axis_name="s")

    @functools.partial(
        pl.kernel,
        mesh=mesh,
        out_type=jax.ShapeDtypeStruct((N // 4,), jnp.int32),
        compiler_params=pltpu.CompilerParams(needs_layout_passes=False),
        scratch_types=[
            pltpu.VMEM((CHUNK,), jnp.int32),
            pltpu.VMEM((WPC,), jnp.int32),
            pltpu.VMEM((256,), jnp.int32),
            pltpu.VMEM((256,), jnp.int32),
            pltpu.VMEM((256,), jnp.int32),
            pltpu.VMEM((256,), jnp.int32),
            pltpu.VMEM((256,), jnp.int32),
        ],
    )
    def k(x_hbm, tbl_hbm, out_hbm, xbuf, obuf, traw, t0, t1, t2, t3):
        wid = lax.axis_index("s") * NC + lax.axis_index("c")

        pltpu.sync_copy(tbl_hbm, traw)
        for kk in range(256 // L):
            sl = pl.ds(kk * L, L)
            v = traw[sl] & 255
            t0[sl] = v
            t1[sl] = v << 8
            t2[sl] = v << 16
            t3[sl] = v << 24

        iota4 = lax.iota(jnp.int32, L) * 4

        def body(i, _):
            base = i * (4 * L)
            g0 = plsc.load_gather(xbuf, [iota4 + base])
            g1 = plsc.load_gather(xbuf, [iota4 + (base + 1)])
            g2 = plsc.load_gather(xbuf, [iota4 + (base + 2)])
            g3 = plsc.load_gather(xbuf, [iota4 + (base + 3)])
            w = (
                plsc.load_gather(t0, [g0])
                | plsc.load_gather(t1, [g1])
                | plsc.load_gather(t2, [g2])
                | plsc.load_gather(t3, [g3])
            )
            obuf[pl.ds(i * L, L)] = w
            return 0

        for c in range(NCHUNK):
            in_off = wid * PER_W + c * CHUNK
            out_off = wid * (PER_W // 4) + c * WPC
            pltpu.sync_copy(x_hbm.at[pl.ds(in_off, CHUNK)], xbuf)
            lax.fori_loop(0, CHUNK // (4 * L), body, 0)
            pltpu.sync_copy(obuf, out_hbm.at[pl.ds(out_off, WPC)])

    return k(xw, tbl32)


def kernel(x, table):
    xw = x.reshape(-1)
    tbl32 = table.astype(jnp.int32)
    words = _sc_lut_call(xw, tbl32)
    y = jax.lax.bitcast_convert_type(words.reshape(ROWS, COLS // 4), jnp.int8)
    return y.reshape(ROWS, COLS)


# speedup vs baseline: 276.1250x; 2.0525x over previous
import functools

import jax
import jax.numpy as jnp
from jax import lax
from jax.experimental import pallas as pl
from jax.experimental.pallas import tpu as pltpu
from jax.experimental.pallas import tpu_sc as plsc

ROWS, COLS = 16384, 200
NC, NS, L = 2, 16, 16
NW = NC * NS
ROWS_W = ROWS // NW
BR = 128
NCHUNK = ROWS_W // BR
COL_BASES = (0, 16, 32, 48, 64, 80, 96, 112, 128, 144, 160, 176, 184)


def _sc_lut_call(x, tbl32):
    mesh = plsc.VectorSubcoreMesh(core_axis_name="c", subcore_axis_name="s")

    @functools.partial(
        pl.kernel,
        mesh=mesh,
        out_type=jax.ShapeDtypeStruct((ROWS, COLS), jnp.int8),
        compiler_params=pltpu.CompilerParams(
            needs_layout_passes=False,
            disable_bounds_checks=True,
        ),
        scratch_types=[
            pltpu.VMEM((BR, COLS), jnp.int32),
            pltpu.VMEM((BR, COLS), jnp.int32),
            pltpu.VMEM((BR, COLS), jnp.int8),
            pltpu.VMEM((BR, COLS), jnp.int8),
            pltpu.VMEM((256,), jnp.int32),
            pltpu.VMEM((256,), jnp.int32),
            pltpu.VMEM((256,), jnp.int32),
            pltpu.VMEM((256,), jnp.int32),
            pltpu.VMEM((256,), jnp.int32),
            pltpu.SemaphoreType.DMA,
            pltpu.SemaphoreType.DMA,
            pltpu.SemaphoreType.DMA,
            pltpu.SemaphoreType.DMA,
        ],
    )
    def k(x_hbm, tbl_hbm, out_hbm, xb0, xb1, ob0, ob1, traw, t0, t1, t2, t3,
          si0, si1, so0, so1):
        wid = lax.axis_index("s") * NC + lax.axis_index("c")
        xbufs, obufs = (xb0, xb1), (ob0, ob1)
        isems, osems = (si0, si1), (so0, so1)

        pltpu.sync_copy(tbl_hbm, traw)
        for kk in range(256 // L):
            sl = pl.ds(kk * L, L)
            v = traw[sl] & 255
            t0[sl] = v
            t1[sl] = v << 8
            t2[sl] = v << 16
            t3[sl] = v << 24

        def in_copy(c, b):
            row0 = wid * ROWS_W + c * BR
            return pltpu.make_async_copy(
                x_hbm.at[pl.ds(row0, BR), :], xbufs[b], isems[b])

        def out_copy(c, b):
            row0 = wid * ROWS_W + c * BR
            return pltpu.make_async_copy(
                obufs[b], out_hbm.at[pl.ds(row0, BR), :], osems[b])

        def make_body(xbuf, obuf):
            def body(s, _):
                r = s * 4
                for c0 in COL_BASES:
                    x0 = xbuf[r, pl.ds(c0, L)]
                    x1 = xbuf[r + 1, pl.ds(c0, L)]
                    x2 = xbuf[r + 2, pl.ds(c0, L)]
                    x3 = xbuf[r + 3, pl.ds(c0, L)]
                    w = (
                        plsc.load_gather(t0, [x0])
                        | plsc.load_gather(t1, [x1])
                        | plsc.load_gather(t2, [x2])
                        | plsc.load_gather(t3, [x3])
                    )
                    obuf[r, pl.ds(jnp.int32(c0), 4 * L)] = plsc.bitcast(
                        w, jnp.int8)
                return 0
            return body

        in_copy(0, 0).start()
        for c in range(NCHUNK):
            b = c % 2
            if c + 1 < NCHUNK:
                in_copy(c + 1, 1 - b).start()
            in_copy(c, b).wait()
            if c >= 2:
                out_copy(c - 2, b).wait()
            lax.fori_loop(0, BR // 4, make_body(xbufs[b], obufs[b]), 0)
            out_copy(c, b).start()
        out_copy(NCHUNK - 2, NCHUNK % 2).wait()
        out_copy(NCHUNK - 1, 1 - NCHUNK % 2).wait()

    return k(x, tbl32)


def kernel(x, table):
    tbl32 = table.astype(jnp.int32)
    return _sc_lut_call(x, tbl32)
